# baseline (speedup 1.0000x reference)
import functools

import jax
import jax.numpy as jnp
from jax import lax
from jax.experimental import pallas as pl
from jax.experimental.pallas import tpu as pltpu

N_NODE = 10000
E_TOT = 320000
C = 128
PW = 16

_NB = 1000
_EB = 512


def _silu(x):
    return x * jax.nn.sigmoid(x)


def _pre_body(h_src_ref, h_tgt_ref, t_emb_ref, w1s_ref, w1t_ref, w1e_ref,
              b1_ref, a_ref, b_ref):
    a_ref[...] = jnp.dot(h_src_ref[...], w1s_ref[...],
                         preferred_element_type=jnp.float32)
    b_ref[...] = (jnp.dot(h_tgt_ref[...], w1t_ref[...],
                          preferred_element_type=jnp.float32)
                  + jnp.dot(t_emb_ref[...], w1e_ref[...],
                            preferred_element_type=jnp.float32)
                  + b1_ref[...])


def _node_pre(h_src, h_tgt, t_emb, w1s, w1t, w1e, b1):
    grid = N_NODE // _NB
    return pl.pallas_call(
        _pre_body,
        grid=(grid,),
        in_specs=[
            pl.BlockSpec((_NB, C), lambda i: (i, 0)),
            pl.BlockSpec((_NB, C), lambda i: (i, 0)),
            pl.BlockSpec((_NB, 32), lambda i: (i, 0)),
            pl.BlockSpec((C, 2 * C), lambda i: (0, 0)),
            pl.BlockSpec((C, 2 * C), lambda i: (0, 0)),
            pl.BlockSpec((32, 2 * C), lambda i: (0, 0)),
            pl.BlockSpec((1, 2 * C), lambda i: (0, 0)),
        ],
        out_specs=[
            pl.BlockSpec((_NB, 2 * C), lambda i: (i, 0)),
            pl.BlockSpec((_NB, 2 * C), lambda i: (i, 0)),
        ],
        out_shape=[
            jax.ShapeDtypeStruct((N_NODE, 2 * C), jnp.float32),
            jax.ShapeDtypeStruct((N_NODE, 2 * C), jnp.float32),
        ],
    )(h_src, h_tgt, t_emb, w1s, w1t, w1e, b1)


def _edge_body(a_ref, b_ref, ps_ref, pt_ref, w1d_ref, w2_ref, b2_ref,
               wc1_ref, bc1_ref, wc2_ref, bc2_ref, msg_ref, wrel_ref):
    rel = pt_ref[...] - ps_ref[...]
    sqd = jnp.sum(rel * rel, axis=1, keepdims=True)
    pre = a_ref[...] + b_ref[...] + sqd * w1d_ref[...]
    h1 = _silu(pre)
    msg = _silu(jnp.dot(h1, w2_ref[...], preferred_element_type=jnp.float32)
                + b2_ref[...])
    t1 = _silu(jnp.dot(msg, wc1_ref[...], preferred_element_type=jnp.float32)
               + bc1_ref[...])
    w = jnp.tanh(jnp.dot(t1, wc2_ref[...], preferred_element_type=jnp.float32)
                 + bc2_ref[...])
    msg_ref[...] = msg
    wrel_ref[...] = w * rel


def _edge_mlp(a_rows, b_rows, ps_rows, pt_rows, w1d, W2, b2, Wc1, bc1, Wc2, bc2):
    grid = E_TOT // _EB
    return pl.pallas_call(
        _edge_body,
        grid=(grid,),
        in_specs=[
            pl.BlockSpec((_EB, 2 * C), lambda i: (i, 0)),
            pl.BlockSpec((_EB, 2 * C), lambda i: (i, 0)),
            pl.BlockSpec((_EB, PW), lambda i: (i, 0)),
            pl.BlockSpec((_EB, PW), lambda i: (i, 0)),
            pl.BlockSpec((1, 2 * C), lambda i: (0, 0)),
            pl.BlockSpec((2 * C, C), lambda i: (0, 0)),
            pl.BlockSpec((1, C), lambda i: (0, 0)),
            pl.BlockSpec((C, C // 2), lambda i: (0, 0)),
            pl.BlockSpec((1, C // 2), lambda i: (0, 0)),
            pl.BlockSpec((C // 2, 1), lambda i: (0, 0)),
            pl.BlockSpec((1, 1), lambda i: (0, 0)),
        ],
        out_specs=[
            pl.BlockSpec((_EB, C), lambda i: (i, 0)),
            pl.BlockSpec((_EB, PW), lambda i: (i, 0)),
        ],
        out_shape=[
            jax.ShapeDtypeStruct((E_TOT, C), jnp.float32),
            jax.ShapeDtypeStruct((E_TOT, PW), jnp.float32),
        ],
    )(a_rows, b_rows, ps_rows, pt_rows, w1d, W2, b2, Wc1, bc1, Wc2, bc2)


def _upd_body(h_ref, agg0_ref, agg1_ref, vel0_ref, vel1_ref, wu1a_ref,
              wu1b_ref, bu1_ref, wu2_ref, bu2_ref, g_ref, bt_ref,
              h_out_ref, vel_out_ref):
    h = h_ref[...]
    agg = agg0_ref[...] + agg1_ref[...]
    u1 = _silu(jnp.dot(h, wu1a_ref[...], preferred_element_type=jnp.float32)
               + jnp.dot(agg, wu1b_ref[...], preferred_element_type=jnp.float32)
               + bu1_ref[...])
    upd = jnp.dot(u1, wu2_ref[...], preferred_element_type=jnp.float32) + bu2_ref[...]
    x = h + upd
    mu = jnp.mean(x, axis=1, keepdims=True)
    xc = x - mu
    var = jnp.mean(xc * xc, axis=1, keepdims=True)
    h_out_ref[...] = xc * lax.rsqrt(var + 1e-5) * g_ref[...] + bt_ref[...]
    vel_out_ref[...] = vel0_ref[...] + vel1_ref[...]


def _node_update(h_tgt, agg0, agg1, vel0, vel1, Wu1a, Wu1b, bu1, Wu2, bu2,
                 gamma, beta):
    grid = N_NODE // _NB
    return pl.pallas_call(
        _upd_body,
        grid=(grid,),
        in_specs=[
            pl.BlockSpec((_NB, C), lambda i: (i, 0)),
            pl.BlockSpec((_NB, C), lambda i: (i, 0)),
            pl.BlockSpec((_NB, C), lambda i: (i, 0)),
            pl.BlockSpec((_NB, PW), lambda i: (i, 0)),
            pl.BlockSpec((_NB, PW), lambda i: (i, 0)),
            pl.BlockSpec((C, C), lambda i: (0, 0)),
            pl.BlockSpec((C, C), lambda i: (0, 0)),
            pl.BlockSpec((1, C), lambda i: (0, 0)),
            pl.BlockSpec((C, C), lambda i: (0, 0)),
            pl.BlockSpec((1, C), lambda i: (0, 0)),
            pl.BlockSpec((1, C), lambda i: (0, 0)),
            pl.BlockSpec((1, C), lambda i: (0, 0)),
        ],
        out_specs=[
            pl.BlockSpec((_NB, C), lambda i: (i, 0)),
            pl.BlockSpec((_NB, PW), lambda i: (i, 0)),
        ],
        out_shape=[
            jax.ShapeDtypeStruct((N_NODE, C), jnp.float32),
            jax.ShapeDtypeStruct((N_NODE, PW), jnp.float32),
        ],
    )(h_tgt, agg0, agg1, vel0, vel1, Wu1a, Wu1b, bu1, Wu2, bu2, gamma, beta)


def kernel(h_src, h_tgt, pos_src, pos_tgt, t_emb_tgt, edge_src, edge_tgt,
           W1, b1, W2, b2, Wc1, bc1, Wc2, bc2, Wu1, bu1, Wu2, bu2,
           gamma, beta):
    w1s = W1[0:C]
    w1t = W1[C:2 * C]
    w1d = W1[2 * C:2 * C + 1]
    w1e = W1[2 * C + 1:]
    A, B = _node_pre(h_src, h_tgt, t_emb_tgt, w1s, w1t, w1e,
                     b1.reshape(1, -1))

    ps_pad = jnp.pad(pos_src, ((0, 0), (0, PW - 3)))
    pt_pad = jnp.pad(pos_tgt, ((0, 0), (0, PW - 3)))

    a_rows = jnp.take(A, edge_src, axis=0)
    b_rows = jnp.take(B, edge_tgt, axis=0)
    ps_rows = jnp.take(ps_pad, edge_src, axis=0)
    pt_rows = jnp.take(pt_pad, edge_tgt, axis=0)

    msg, wrel = _edge_mlp(a_rows, b_rows, ps_rows, pt_rows, w1d, W2,
                          b2.reshape(1, -1), Wc1, bc1.reshape(1, -1), Wc2,
                          bc2.reshape(1, 1))

    agg = jnp.zeros((N_NODE, C), jnp.float32).at[edge_tgt].add(msg)
    vel = jnp.zeros((N_NODE, PW), jnp.float32).at[edge_tgt].add(wrel)
    zn = jnp.zeros((N_NODE, C), jnp.float32)
    zv = jnp.zeros((N_NODE, PW), jnp.float32)

    h_new, vel_out = _node_update(h_tgt, agg, zn, vel, zv, Wu1[:C], Wu1[C:],
                                  bu1.reshape(1, -1), Wu2, bu2.reshape(1, -1),
                                  gamma.reshape(1, -1), beta.reshape(1, -1))
    return h_new, vel_out[:, :3]


# speedup vs baseline: 2.1236x; 2.1236x over previous
import functools

import jax
import jax.numpy as jnp
from jax import lax
from jax.experimental import pallas as pl
from jax.experimental.pallas import tpu as pltpu
from jax.experimental.pallas import tpu_sc as plsc

N_NODE = 10000
E_TOT = 320000
C = 128
PW = 16

_NB = 1000
_EB = 512

_NW = 32
_EPW = E_TOT // _NW
_CH = 80
_NCH = _EPW // _CH

_sc_mesh = functools.partial(
    plsc.VectorSubcoreMesh, core_axis_name="c", subcore_axis_name="s")


def _gather_body(a_hbm, b_hbm, ps_hbm, pt_hbm, es_hbm, et_hbm,
                 a_out, b_out, ps_out, pt_out,
                 idx_s, idx_t, abuf, bbuf, psbuf, ptbuf, sem):
    wid = lax.axis_index("s") * 2 + lax.axis_index("c")

    def body(i, carry):
        base = wid * _EPW + i * _CH
        pltpu.sync_copy(es_hbm.at[pl.ds(base, _CH)], idx_s)
        pltpu.sync_copy(et_hbm.at[pl.ds(base, _CH)], idx_t)
        c1 = pltpu.async_copy(a_hbm.at[idx_s], abuf, sem)
        c2 = pltpu.async_copy(b_hbm.at[idx_t], bbuf, sem)
        c3 = pltpu.async_copy(ps_hbm.at[idx_s], psbuf, sem)
        c4 = pltpu.async_copy(pt_hbm.at[idx_t], ptbuf, sem)
        c1.wait(); c2.wait(); c3.wait(); c4.wait()
        pltpu.sync_copy(abuf, a_out.at[pl.ds(base, _CH)])
        pltpu.sync_copy(bbuf, b_out.at[pl.ds(base, _CH)])
        pltpu.sync_copy(psbuf, ps_out.at[pl.ds(base, _CH)])
        pltpu.sync_copy(ptbuf, pt_out.at[pl.ds(base, _CH)])
        return carry

    lax.fori_loop(0, _NCH, body, 0)


def _sc_gather(A, B, ps_pad, pt_pad, edge_src, edge_tgt):
    f = pl.kernel(
        _gather_body, mesh=_sc_mesh(),
        compiler_params=pltpu.CompilerParams(use_tc_tiling_on_sc=False),
        out_type=[
            jax.ShapeDtypeStruct((E_TOT, 2 * C), jnp.float32),
            jax.ShapeDtypeStruct((E_TOT, 2 * C), jnp.float32),
            jax.ShapeDtypeStruct((E_TOT, PW), jnp.float32),
            jax.ShapeDtypeStruct((E_TOT, PW), jnp.float32),
        ],
        scratch_types=[
            pltpu.VMEM((_CH,), jnp.int32),
            pltpu.VMEM((_CH,), jnp.int32),
            pltpu.VMEM((_CH, 2 * C), jnp.float32),
            pltpu.VMEM((_CH, 2 * C), jnp.float32),
            pltpu.VMEM((_CH, PW), jnp.float32),
            pltpu.VMEM((_CH, PW), jnp.float32),
            pltpu.SemaphoreType.DMA,
        ],
    )
    return f(A, B, ps_pad, pt_pad, edge_src, edge_tgt)


def _silu(x):
    return x * jax.nn.sigmoid(x)


def _scatter_body(msg_hbm, wrel_hbm, et_hbm, zn_hbm, zv_hbm,
                  agg_out, vel_out,
                  idx_t, mbuf, wbuf, agg_acc, vel_acc, sem):
    c = lax.axis_index("c")
    s = lax.axis_index("s")
    wid = s * 2 + c

    @pl.when(s == 0)
    def _init():
        pltpu.sync_copy(zn_hbm, agg_acc)
        pltpu.sync_copy(zv_hbm, vel_acc)

    plsc.subcore_barrier()

    def body(i, carry):
        base = wid * _EPW + i * _CH
        pltpu.sync_copy(et_hbm.at[pl.ds(base, _CH)], idx_t)
        c1 = pltpu.async_copy(msg_hbm.at[pl.ds(base, _CH)], mbuf, sem)
        c2 = pltpu.async_copy(wrel_hbm.at[pl.ds(base, _CH)], wbuf, sem)
        c1.wait(); c2.wait()
        pltpu.sync_copy(mbuf, agg_acc.at[idx_t], add=True)
        pltpu.sync_copy(wbuf, vel_acc.at[idx_t], add=True)
        return carry

    lax.fori_loop(0, _NCH, body, 0)
    plsc.subcore_barrier()

    rows = N_NODE // 16
    rbase = s * rows
    pltpu.sync_copy(agg_acc.at[pl.ds(rbase, rows)],
                    agg_out.at[c].at[pl.ds(rbase, rows)])
    pltpu.sync_copy(vel_acc.at[pl.ds(rbase, rows)],
                    vel_out.at[c].at[pl.ds(rbase, rows)])


def _sc_scatter(msg, wrel, edge_tgt, zn, zv):
    f = pl.kernel(
        _scatter_body, mesh=_sc_mesh(),
        compiler_params=pltpu.CompilerParams(use_tc_tiling_on_sc=False),
        out_type=[
            jax.ShapeDtypeStruct((2, N_NODE, C), jnp.float32),
            jax.ShapeDtypeStruct((2, N_NODE, PW), jnp.float32),
        ],
        scratch_types=[
            pltpu.VMEM((_CH,), jnp.int32),
            pltpu.VMEM((_CH, C), jnp.float32),
            pltpu.VMEM((_CH, PW), jnp.float32),
            pltpu.VMEM_SHARED((N_NODE, C), jnp.float32),
            pltpu.VMEM_SHARED((N_NODE, PW), jnp.float32),
            pltpu.SemaphoreType.DMA,
        ],
    )
    return f(msg, wrel, edge_tgt, zn, zv)


def _pre_body(h_src_ref, h_tgt_ref, t_emb_ref, w1s_ref, w1t_ref, w1e_ref,
              b1_ref, a_ref, b_ref):
    a_ref[...] = jnp.dot(h_src_ref[...], w1s_ref[...],
                         preferred_element_type=jnp.float32)
    b_ref[...] = (jnp.dot(h_tgt_ref[...], w1t_ref[...],
                          preferred_element_type=jnp.float32)
                  + jnp.dot(t_emb_ref[...], w1e_ref[...],
                            preferred_element_type=jnp.float32)
                  + b1_ref[...])


def _node_pre(h_src, h_tgt, t_emb, w1s, w1t, w1e, b1):
    grid = N_NODE // _NB
    return pl.pallas_call(
        _pre_body,
        grid=(grid,),
        in_specs=[
            pl.BlockSpec((_NB, C), lambda i: (i, 0)),
            pl.BlockSpec((_NB, C), lambda i: (i, 0)),
            pl.BlockSpec((_NB, 32), lambda i: (i, 0)),
            pl.BlockSpec((C, 2 * C), lambda i: (0, 0)),
            pl.BlockSpec((C, 2 * C), lambda i: (0, 0)),
            pl.BlockSpec((32, 2 * C), lambda i: (0, 0)),
            pl.BlockSpec((1, 2 * C), lambda i: (0, 0)),
        ],
        out_specs=[
            pl.BlockSpec((_NB, 2 * C), lambda i: (i, 0)),
            pl.BlockSpec((_NB, 2 * C), lambda i: (i, 0)),
        ],
        out_shape=[
            jax.ShapeDtypeStruct((N_NODE, 2 * C), jnp.float32),
            jax.ShapeDtypeStruct((N_NODE, 2 * C), jnp.float32),
        ],
    )(h_src, h_tgt, t_emb, w1s, w1t, w1e, b1)


def _edge_body(a_ref, b_ref, ps_ref, pt_ref, w1d_ref, w2_ref, b2_ref,
               wc1_ref, bc1_ref, wc2_ref, bc2_ref, msg_ref, wrel_ref):
    rel = pt_ref[...] - ps_ref[...]
    sqd = jnp.sum(rel * rel, axis=1, keepdims=True)
    pre = a_ref[...] + b_ref[...] + sqd * w1d_ref[...]
    h1 = _silu(pre)
    msg = _silu(jnp.dot(h1, w2_ref[...], preferred_element_type=jnp.float32)
                + b2_ref[...])
    t1 = _silu(jnp.dot(msg, wc1_ref[...], preferred_element_type=jnp.float32)
               + bc1_ref[...])
    w = jnp.tanh(jnp.dot(t1, wc2_ref[...], preferred_element_type=jnp.float32)
                 + bc2_ref[...])
    msg_ref[...] = msg
    wrel_ref[...] = w * rel


def _edge_mlp(a_rows, b_rows, ps_rows, pt_rows, w1d, W2, b2, Wc1, bc1, Wc2, bc2):
    grid = E_TOT // _EB
    return pl.pallas_call(
        _edge_body,
        grid=(grid,),
        in_specs=[
            pl.BlockSpec((_EB, 2 * C), lambda i: (i, 0)),
            pl.BlockSpec((_EB, 2 * C), lambda i: (i, 0)),
            pl.BlockSpec((_EB, PW), lambda i: (i, 0)),
            pl.BlockSpec((_EB, PW), lambda i: (i, 0)),
            pl.BlockSpec((1, 2 * C), lambda i: (0, 0)),
            pl.BlockSpec((2 * C, C), lambda i: (0, 0)),
            pl.BlockSpec((1, C), lambda i: (0, 0)),
            pl.BlockSpec((C, C // 2), lambda i: (0, 0)),
            pl.BlockSpec((1, C // 2), lambda i: (0, 0)),
            pl.BlockSpec((C // 2, 1), lambda i: (0, 0)),
            pl.BlockSpec((1, 1), lambda i: (0, 0)),
        ],
        out_specs=[
            pl.BlockSpec((_EB, C), lambda i: (i, 0)),
            pl.BlockSpec((_EB, PW), lambda i: (i, 0)),
        ],
        out_shape=[
            jax.ShapeDtypeStruct((E_TOT, C), jnp.float32),
            jax.ShapeDtypeStruct((E_TOT, PW), jnp.float32),
        ],
    )(a_rows, b_rows, ps_rows, pt_rows, w1d, W2, b2, Wc1, bc1, Wc2, bc2)


def _upd_body(h_ref, agg0_ref, agg1_ref, vel0_ref, vel1_ref, wu1a_ref,
              wu1b_ref, bu1_ref, wu2_ref, bu2_ref, g_ref, bt_ref,
              h_out_ref, vel_out_ref):
    h = h_ref[...]
    agg = agg0_ref[...] + agg1_ref[...]
    u1 = _silu(jnp.dot(h, wu1a_ref[...], preferred_element_type=jnp.float32)
               + jnp.dot(agg, wu1b_ref[...], preferred_element_type=jnp.float32)
               + bu1_ref[...])
    upd = jnp.dot(u1, wu2_ref[...], preferred_element_type=jnp.float32) + bu2_ref[...]
    x = h + upd
    mu = jnp.mean(x, axis=1, keepdims=True)
    xc = x - mu
    var = jnp.mean(xc * xc, axis=1, keepdims=True)
    h_out_ref[...] = xc * lax.rsqrt(var + 1e-5) * g_ref[...] + bt_ref[...]
    vel_out_ref[...] = vel0_ref[...] + vel1_ref[...]


def _node_update(h_tgt, agg0, agg1, vel0, vel1, Wu1a, Wu1b, bu1, Wu2, bu2,
                 gamma, beta):
    grid = N_NODE // _NB
    return pl.pallas_call(
        _upd_body,
        grid=(grid,),
        in_specs=[
            pl.BlockSpec((_NB, C), lambda i: (i, 0)),
            pl.BlockSpec((_NB, C), lambda i: (i, 0)),
            pl.BlockSpec((_NB, C), lambda i: (i, 0)),
            pl.BlockSpec((_NB, PW), lambda i: (i, 0)),
            pl.BlockSpec((_NB, PW), lambda i: (i, 0)),
            pl.BlockSpec((C, C), lambda i: (0, 0)),
            pl.BlockSpec((C, C), lambda i: (0, 0)),
            pl.BlockSpec((1, C), lambda i: (0, 0)),
            pl.BlockSpec((C, C), lambda i: (0, 0)),
            pl.BlockSpec((1, C), lambda i: (0, 0)),
            pl.BlockSpec((1, C), lambda i: (0, 0)),
            pl.BlockSpec((1, C), lambda i: (0, 0)),
        ],
        out_specs=[
            pl.BlockSpec((_NB, C), lambda i: (i, 0)),
            pl.BlockSpec((_NB, PW), lambda i: (i, 0)),
        ],
        out_shape=[
            jax.ShapeDtypeStruct((N_NODE, C), jnp.float32),
            jax.ShapeDtypeStruct((N_NODE, PW), jnp.float32),
        ],
    )(h_tgt, agg0, agg1, vel0, vel1, Wu1a, Wu1b, bu1, Wu2, bu2, gamma, beta)


def kernel(h_src, h_tgt, pos_src, pos_tgt, t_emb_tgt, edge_src, edge_tgt,
           W1, b1, W2, b2, Wc1, bc1, Wc2, bc2, Wu1, bu1, Wu2, bu2,
           gamma, beta):
    w1s = W1[0:C]
    w1t = W1[C:2 * C]
    w1d = W1[2 * C:2 * C + 1]
    w1e = W1[2 * C + 1:]
    A, B = _node_pre(h_src, h_tgt, t_emb_tgt, w1s, w1t, w1e,
                     b1.reshape(1, -1))

    ps_pad = jnp.pad(pos_src, ((0, 0), (0, PW - 3)))
    pt_pad = jnp.pad(pos_tgt, ((0, 0), (0, PW - 3)))

    a_rows, b_rows, ps_rows, pt_rows = _sc_gather(
        A, B, ps_pad, pt_pad, edge_src, edge_tgt)

    msg, wrel = _edge_mlp(a_rows, b_rows, ps_rows, pt_rows, w1d, W2,
                          b2.reshape(1, -1), Wc1, bc1.reshape(1, -1), Wc2,
                          bc2.reshape(1, 1))

    zn = jnp.zeros((N_NODE, C), jnp.float32)
    zv = jnp.zeros((N_NODE, PW), jnp.float32)
    agg2, vel2 = _sc_scatter(msg, wrel, edge_tgt, zn, zv)

    h_new, vel_out = _node_update(h_tgt, agg2[0], agg2[1], vel2[0], vel2[1],
                                  Wu1[:C], Wu1[C:],
                                  bu1.reshape(1, -1), Wu2, bu2.reshape(1, -1),
                                  gamma.reshape(1, -1), beta.reshape(1, -1))
    return h_new, vel_out[:, :3]


# speedup vs baseline: 2.3635x; 1.1130x over previous
import functools

import jax
import jax.numpy as jnp
from jax import lax
from jax.experimental import pallas as pl
from jax.experimental.pallas import tpu as pltpu
from jax.experimental.pallas import tpu_sc as plsc

N_NODE = 10000
E_TOT = 320000
C = 128
PW = 16

_NB = 1000
_EB = 512

_NW = 32
_EPW = E_TOT // _NW
_CH = 80
_NCH = _EPW // _CH

_sc_mesh = functools.partial(
    plsc.VectorSubcoreMesh, core_axis_name="c", subcore_axis_name="s")


def _gather_body(a_hbm, b_hbm, ps_hbm, pt_hbm, es_hbm, et_hbm,
                 a_out, b_out, ps_out, pt_out,
                 bufs0, bufs1, gsem0, gsem1, osem0, osem1):
    wid = lax.axis_index("s") * 2 + lax.axis_index("c")
    tbase = wid * _EPW

    def fire_g(base, bufs, sem):
        idx_s, idx_t, ab, bb, psb, ptb = bufs
        pltpu.sync_copy(es_hbm.at[pl.ds(base, _CH)], idx_s)
        pltpu.sync_copy(et_hbm.at[pl.ds(base, _CH)], idx_t)
        pltpu.async_copy(a_hbm.at[idx_s], ab, sem)
        pltpu.async_copy(b_hbm.at[idx_t], bb, sem)
        pltpu.async_copy(ps_hbm.at[idx_s], psb, sem)
        pltpu.async_copy(pt_hbm.at[idx_t], ptb, sem)

    def wait_g(bufs, sem):
        _, _, ab, bb, psb, ptb = bufs
        pltpu.make_async_copy(a_hbm.at[pl.ds(0, _CH)], ab, sem).wait()
        pltpu.make_async_copy(b_hbm.at[pl.ds(0, _CH)], bb, sem).wait()
        pltpu.make_async_copy(ps_hbm.at[pl.ds(0, _CH)], psb, sem).wait()
        pltpu.make_async_copy(pt_hbm.at[pl.ds(0, _CH)], ptb, sem).wait()

    def fire_o(base, bufs, sem):
        _, _, ab, bb, psb, ptb = bufs
        pltpu.async_copy(ab, a_out.at[pl.ds(base, _CH)], sem)
        pltpu.async_copy(bb, b_out.at[pl.ds(base, _CH)], sem)
        pltpu.async_copy(psb, ps_out.at[pl.ds(base, _CH)], sem)
        pltpu.async_copy(ptb, pt_out.at[pl.ds(base, _CH)], sem)

    def wait_o(bufs, sem):
        _, _, ab, bb, psb, ptb = bufs
        pltpu.make_async_copy(ab, a_out.at[pl.ds(0, _CH)], sem).wait()
        pltpu.make_async_copy(bb, b_out.at[pl.ds(0, _CH)], sem).wait()
        pltpu.make_async_copy(psb, ps_out.at[pl.ds(0, _CH)], sem).wait()
        pltpu.make_async_copy(ptb, pt_out.at[pl.ds(0, _CH)], sem).wait()

    fire_g(tbase, bufs0, gsem0)

    def body(j, carry):
        base0 = tbase + (2 * j) * _CH
        base1 = base0 + _CH
        base2 = base0 + 2 * _CH

        @pl.when(j > 0)
        def _():
            wait_o(bufs1, osem1)

        @pl.when(2 * j + 1 < _NCH)
        def _():
            fire_g(base1, bufs1, gsem1)

        wait_g(bufs0, gsem0)
        fire_o(base0, bufs0, osem0)

        @pl.when(2 * j + 2 < _NCH)
        def _():
            wait_o(bufs0, osem0)
            fire_g(base2, bufs0, gsem0)

        @pl.when(2 * j + 1 < _NCH)
        def _():
            wait_g(bufs1, gsem1)
            fire_o(base1, bufs1, osem1)

        return carry

    lax.fori_loop(0, (_NCH + 1) // 2, body, 0)
    wait_o(bufs0, osem0)


def _sc_gather(A, B, ps_pad, pt_pad, edge_src, edge_tgt):
    assert _NCH % 2 == 1
    bufset = lambda: [
        pltpu.VMEM((_CH,), jnp.int32),
        pltpu.VMEM((_CH,), jnp.int32),
        pltpu.VMEM((_CH, 2 * C), jnp.float32),
        pltpu.VMEM((_CH, 2 * C), jnp.float32),
        pltpu.VMEM((_CH, PW), jnp.float32),
        pltpu.VMEM((_CH, PW), jnp.float32),
    ]
    f = pl.kernel(
        _gather_body, mesh=_sc_mesh(),
        compiler_params=pltpu.CompilerParams(use_tc_tiling_on_sc=False),
        out_type=[
            jax.ShapeDtypeStruct((E_TOT, 2 * C), jnp.float32),
            jax.ShapeDtypeStruct((E_TOT, 2 * C), jnp.float32),
            jax.ShapeDtypeStruct((E_TOT, PW), jnp.float32),
            jax.ShapeDtypeStruct((E_TOT, PW), jnp.float32),
        ],
        scratch_types=[
            bufset(), bufset(),
            pltpu.SemaphoreType.DMA, pltpu.SemaphoreType.DMA,
            pltpu.SemaphoreType.DMA, pltpu.SemaphoreType.DMA,
        ],
    )
    return f(A, B, ps_pad, pt_pad, edge_src, edge_tgt)


def _silu(x):
    return x * jax.nn.sigmoid(x)


def _scatter_body(msg_hbm, wrel_hbm, et_hbm, zn_hbm, zv_hbm,
                  agg_out, vel_out,
                  bufs0, bufs1, agg_acc, vel_acc, lsem0, lsem1):
    c = lax.axis_index("c")
    s = lax.axis_index("s")
    wid = s * 2 + c
    tbase = wid * _EPW

    @pl.when(s == 0)
    def _init():
        pltpu.sync_copy(zn_hbm, agg_acc)
        pltpu.sync_copy(zv_hbm, vel_acc)

    def fire_l(base, bufs, sem):
        idx_t, mb, wb = bufs
        pltpu.sync_copy(et_hbm.at[pl.ds(base, _CH)], idx_t)
        pltpu.async_copy(msg_hbm.at[pl.ds(base, _CH)], mb, sem)
        pltpu.async_copy(wrel_hbm.at[pl.ds(base, _CH)], wb, sem)

    def wait_l(bufs, sem):
        _, mb, wb = bufs
        pltpu.make_async_copy(msg_hbm.at[pl.ds(0, _CH)], mb, sem).wait()
        pltpu.make_async_copy(wrel_hbm.at[pl.ds(0, _CH)], wb, sem).wait()

    def scat(bufs):
        idx_t, mb, wb = bufs
        pltpu.sync_copy(mb, agg_acc.at[idx_t], add=True)
        pltpu.sync_copy(wb, vel_acc.at[idx_t], add=True)

    plsc.subcore_barrier()
    fire_l(tbase, bufs0, lsem0)

    def body(j, carry):
        base1 = tbase + (2 * j + 1) * _CH
        base2 = tbase + (2 * j + 2) * _CH

        @pl.when(2 * j + 1 < _NCH)
        def _():
            fire_l(base1, bufs1, lsem1)

        wait_l(bufs0, lsem0)
        scat(bufs0)

        @pl.when(2 * j + 2 < _NCH)
        def _():
            fire_l(base2, bufs0, lsem0)

        @pl.when(2 * j + 1 < _NCH)
        def _():
            wait_l(bufs1, lsem1)
            scat(bufs1)

        return carry

    lax.fori_loop(0, (_NCH + 1) // 2, body, 0)
    plsc.subcore_barrier()

    rows = N_NODE // 16
    rbase = s * rows
    pltpu.sync_copy(agg_acc.at[pl.ds(rbase, rows)],
                    agg_out.at[c].at[pl.ds(rbase, rows)])
    pltpu.sync_copy(vel_acc.at[pl.ds(rbase, rows)],
                    vel_out.at[c].at[pl.ds(rbase, rows)])


def _sc_scatter(msg, wrel, edge_tgt, zn, zv):
    f = pl.kernel(
        _scatter_body, mesh=_sc_mesh(),
        compiler_params=pltpu.CompilerParams(use_tc_tiling_on_sc=False),
        out_type=[
            jax.ShapeDtypeStruct((2, N_NODE, C), jnp.float32),
            jax.ShapeDtypeStruct((2, N_NODE, PW), jnp.float32),
        ],
        scratch_types=[
            [
                pltpu.VMEM((_CH,), jnp.int32),
                pltpu.VMEM((_CH, C), jnp.float32),
                pltpu.VMEM((_CH, PW), jnp.float32),
            ],
            [
                pltpu.VMEM((_CH,), jnp.int32),
                pltpu.VMEM((_CH, C), jnp.float32),
                pltpu.VMEM((_CH, PW), jnp.float32),
            ],
            pltpu.VMEM_SHARED((N_NODE, C), jnp.float32),
            pltpu.VMEM_SHARED((N_NODE, PW), jnp.float32),
            pltpu.SemaphoreType.DMA, pltpu.SemaphoreType.DMA,
        ],
    )
    return f(msg, wrel, edge_tgt, zn, zv)


def _pre_body(h_src_ref, h_tgt_ref, t_emb_ref, w1s_ref, w1t_ref, w1e_ref,
              b1_ref, a_ref, b_ref):
    a_ref[...] = jnp.dot(h_src_ref[...], w1s_ref[...],
                         preferred_element_type=jnp.float32)
    b_ref[...] = (jnp.dot(h_tgt_ref[...], w1t_ref[...],
                          preferred_element_type=jnp.float32)
                  + jnp.dot(t_emb_ref[...], w1e_ref[...],
                            preferred_element_type=jnp.float32)
                  + b1_ref[...])


def _node_pre(h_src, h_tgt, t_emb, w1s, w1t, w1e, b1):
    grid = N_NODE // _NB
    return pl.pallas_call(
        _pre_body,
        grid=(grid,),
        in_specs=[
            pl.BlockSpec((_NB, C), lambda i: (i, 0)),
            pl.BlockSpec((_NB, C), lambda i: (i, 0)),
            pl.BlockSpec((_NB, 32), lambda i: (i, 0)),
            pl.BlockSpec((C, 2 * C), lambda i: (0, 0)),
            pl.BlockSpec((C, 2 * C), lambda i: (0, 0)),
            pl.BlockSpec((32, 2 * C), lambda i: (0, 0)),
            pl.BlockSpec((1, 2 * C), lambda i: (0, 0)),
        ],
        out_specs=[
            pl.BlockSpec((_NB, 2 * C), lambda i: (i, 0)),
            pl.BlockSpec((_NB, 2 * C), lambda i: (i, 0)),
        ],
        out_shape=[
            jax.ShapeDtypeStruct((N_NODE, 2 * C), jnp.float32),
            jax.ShapeDtypeStruct((N_NODE, 2 * C), jnp.float32),
        ],
    )(h_src, h_tgt, t_emb, w1s, w1t, w1e, b1)


def _edge_body(a_ref, b_ref, ps_ref, pt_ref, w1d_ref, w2_ref, b2_ref,
               wc1_ref, bc1_ref, wc2_ref, bc2_ref, msg_ref, wrel_ref):
    rel = pt_ref[...] - ps_ref[...]
    sqd = jnp.sum(rel * rel, axis=1, keepdims=True)
    pre = a_ref[...] + b_ref[...] + sqd * w1d_ref[...]
    h1 = _silu(pre)
    msg = _silu(jnp.dot(h1, w2_ref[...], preferred_element_type=jnp.float32)
                + b2_ref[...])
    t1 = _silu(jnp.dot(msg, wc1_ref[...], preferred_element_type=jnp.float32)
               + bc1_ref[...])
    w = jnp.tanh(jnp.dot(t1, wc2_ref[...], preferred_element_type=jnp.float32)
                 + bc2_ref[...])
    msg_ref[...] = msg
    wrel_ref[...] = w * rel


def _edge_mlp(a_rows, b_rows, ps_rows, pt_rows, w1d, W2, b2, Wc1, bc1, Wc2, bc2):
    grid = E_TOT // _EB
    return pl.pallas_call(
        _edge_body,
        grid=(grid,),
        in_specs=[
            pl.BlockSpec((_EB, 2 * C), lambda i: (i, 0)),
            pl.BlockSpec((_EB, 2 * C), lambda i: (i, 0)),
            pl.BlockSpec((_EB, PW), lambda i: (i, 0)),
            pl.BlockSpec((_EB, PW), lambda i: (i, 0)),
            pl.BlockSpec((1, 2 * C), lambda i: (0, 0)),
            pl.BlockSpec((2 * C, C), lambda i: (0, 0)),
            pl.BlockSpec((1, C), lambda i: (0, 0)),
            pl.BlockSpec((C, C // 2), lambda i: (0, 0)),
            pl.BlockSpec((1, C // 2), lambda i: (0, 0)),
            pl.BlockSpec((C // 2, 1), lambda i: (0, 0)),
            pl.BlockSpec((1, 1), lambda i: (0, 0)),
        ],
        out_specs=[
            pl.BlockSpec((_EB, C), lambda i: (i, 0)),
            pl.BlockSpec((_EB, PW), lambda i: (i, 0)),
        ],
        out_shape=[
            jax.ShapeDtypeStruct((E_TOT, C), jnp.float32),
            jax.ShapeDtypeStruct((E_TOT, PW), jnp.float32),
        ],
    )(a_rows, b_rows, ps_rows, pt_rows, w1d, W2, b2, Wc1, bc1, Wc2, bc2)


def _upd_body(h_ref, agg0_ref, agg1_ref, vel0_ref, vel1_ref, wu1a_ref,
              wu1b_ref, bu1_ref, wu2_ref, bu2_ref, g_ref, bt_ref,
              h_out_ref, vel_out_ref):
    h = h_ref[...]
    agg = agg0_ref[...] + agg1_ref[...]
    u1 = _silu(jnp.dot(h, wu1a_ref[...], preferred_element_type=jnp.float32)
               + jnp.dot(agg, wu1b_ref[...], preferred_element_type=jnp.float32)
               + bu1_ref[...])
    upd = jnp.dot(u1, wu2_ref[...], preferred_element_type=jnp.float32) + bu2_ref[...]
    x = h + upd
    mu = jnp.mean(x, axis=1, keepdims=True)
    xc = x - mu
    var = jnp.mean(xc * xc, axis=1, keepdims=True)
    h_out_ref[...] = xc * lax.rsqrt(var + 1e-5) * g_ref[...] + bt_ref[...]
    vel_out_ref[...] = vel0_ref[...] + vel1_ref[...]


def _node_update(h_tgt, agg0, agg1, vel0, vel1, Wu1a, Wu1b, bu1, Wu2, bu2,
                 gamma, beta):
    grid = N_NODE // _NB
    return pl.pallas_call(
        _upd_body,
        grid=(grid,),
        in_specs=[
            pl.BlockSpec((_NB, C), lambda i: (i, 0)),
            pl.BlockSpec((_NB, C), lambda i: (i, 0)),
            pl.BlockSpec((_NB, C), lambda i: (i, 0)),
            pl.BlockSpec((_NB, PW), lambda i: (i, 0)),
            pl.BlockSpec((_NB, PW), lambda i: (i, 0)),
            pl.BlockSpec((C, C), lambda i: (0, 0)),
            pl.BlockSpec((C, C), lambda i: (0, 0)),
            pl.BlockSpec((1, C), lambda i: (0, 0)),
            pl.BlockSpec((C, C), lambda i: (0, 0)),
            pl.BlockSpec((1, C), lambda i: (0, 0)),
            pl.BlockSpec((1, C), lambda i: (0, 0)),
            pl.BlockSpec((1, C), lambda i: (0, 0)),
        ],
        out_specs=[
            pl.BlockSpec((_NB, C), lambda i: (i, 0)),
            pl.BlockSpec((_NB, PW), lambda i: (i, 0)),
        ],
        out_shape=[
            jax.ShapeDtypeStruct((N_NODE, C), jnp.float32),
            jax.ShapeDtypeStruct((N_NODE, PW), jnp.float32),
        ],
    )(h_tgt, agg0, agg1, vel0, vel1, Wu1a, Wu1b, bu1, Wu2, bu2, gamma, beta)


def kernel(h_src, h_tgt, pos_src, pos_tgt, t_emb_tgt, edge_src, edge_tgt,
           W1, b1, W2, b2, Wc1, bc1, Wc2, bc2, Wu1, bu1, Wu2, bu2,
           gamma, beta):
    w1s = W1[0:C]
    w1t = W1[C:2 * C]
    w1d = W1[2 * C:2 * C + 1]
    w1e = W1[2 * C + 1:]
    A, B = _node_pre(h_src, h_tgt, t_emb_tgt, w1s, w1t, w1e,
                     b1.reshape(1, -1))

    ps_pad = jnp.pad(pos_src, ((0, 0), (0, PW - 3)))
    pt_pad = jnp.pad(pos_tgt, ((0, 0), (0, PW - 3)))

    a_rows, b_rows, ps_rows, pt_rows = _sc_gather(
        A, B, ps_pad, pt_pad, edge_src, edge_tgt)

    msg, wrel = _edge_mlp(a_rows, b_rows, ps_rows, pt_rows, w1d, W2,
                          b2.reshape(1, -1), Wc1, bc1.reshape(1, -1), Wc2,
                          bc2.reshape(1, 1))

    zn = jnp.zeros((N_NODE, C), jnp.float32)
    zv = jnp.zeros((N_NODE, PW), jnp.float32)
    agg2, vel2 = _sc_scatter(msg, wrel, edge_tgt, zn, zv)

    h_new, vel_out = _node_update(h_tgt, agg2[0], agg2[1], vel2[0], vel2[1],
                                  Wu1[:C], Wu1[C:],
                                  bu1.reshape(1, -1), Wu2, bu2.reshape(1, -1),
                                  gamma.reshape(1, -1), beta.reshape(1, -1))
    return h_new, vel_out[:, :3]


# speedup vs baseline: 3.8297x; 1.6203x over previous
import functools

import jax
import jax.numpy as jnp
from jax import lax
from jax.experimental import pallas as pl
from jax.experimental.pallas import tpu as pltpu
from jax.experimental.pallas import tpu_sc as plsc

N_NODE = 10000
E_TOT = 320000
C = 128
PW = 16

_NB = 1000
_EB = 512

_NW = 32
_EPW = E_TOT // _NW
_CH = 80
_NCH = _EPW // _CH

_sc_mesh = functools.partial(
    plsc.VectorSubcoreMesh, core_axis_name="c", subcore_axis_name="s")


def _gather_body(a_hbm, b_hbm, es_hbm, et_hbm, a_out, b_out,
                 bufs0, bufs1, gsem0, gsem1, osem0, osem1):
    wid = lax.axis_index("s") * 2 + lax.axis_index("c")
    tbase = wid * _EPW

    def fire_g(base, bufs, gsem):
        idx_s, idx_t, ab, bb = bufs
        pltpu.sync_copy(es_hbm.at[pl.ds(base, _CH)], idx_s)
        pltpu.sync_copy(et_hbm.at[pl.ds(base, _CH)], idx_t)
        pltpu.async_copy(a_hbm.at[idx_s], ab, gsem)
        pltpu.async_copy(b_hbm.at[idx_t], bb, gsem)

    def wait_g(bufs, sem):
        _, _, ab, bb = bufs
        pltpu.make_async_copy(a_hbm.at[pl.ds(0, _CH)], ab, sem).wait()
        pltpu.make_async_copy(b_hbm.at[pl.ds(0, _CH)], bb, sem).wait()

    def fire_o(base, bufs, sem):
        _, _, ab, bb = bufs
        pltpu.async_copy(ab, a_out.at[pl.ds(base, _CH)], sem)
        pltpu.async_copy(bb, b_out.at[pl.ds(base, _CH)], sem)

    def wait_o(bufs, sem):
        _, _, ab, bb = bufs
        pltpu.make_async_copy(ab, a_out.at[pl.ds(0, _CH)], sem).wait()
        pltpu.make_async_copy(bb, b_out.at[pl.ds(0, _CH)], sem).wait()

    fire_g(tbase, bufs0, gsem0)

    def body(j, carry):
        base0 = tbase + (2 * j) * _CH
        base1 = base0 + _CH
        base2 = base0 + 2 * _CH

        @pl.when(j > 0)
        def _():
            wait_o(bufs1, osem1)

        @pl.when(2 * j + 1 < _NCH)
        def _():
            fire_g(base1, bufs1, gsem1)

        wait_g(bufs0, gsem0)
        fire_o(base0, bufs0, osem0)

        @pl.when(2 * j + 2 < _NCH)
        def _():
            wait_o(bufs0, osem0)
            fire_g(base2, bufs0, gsem0)

        @pl.when(2 * j + 1 < _NCH)
        def _():
            wait_g(bufs1, gsem1)
            fire_o(base1, bufs1, osem1)

        return carry

    lax.fori_loop(0, (_NCH + 1) // 2, body, 0)
    wait_o(bufs0, osem0)


def _sc_gather(A, B, edge_src, edge_tgt):
    assert _NCH % 2 == 1
    bufset = lambda: [
        pltpu.VMEM((_CH,), jnp.int32),
        pltpu.VMEM((_CH,), jnp.int32),
        pltpu.VMEM((_CH, 2 * C), jnp.float32),
        pltpu.VMEM((_CH, 2 * C), jnp.float32),
    ]
    f = pl.kernel(
        _gather_body, mesh=_sc_mesh(),
        out_type=[
            jax.ShapeDtypeStruct((E_TOT, 2 * C), jnp.float32),
            jax.ShapeDtypeStruct((E_TOT, 2 * C), jnp.float32),
        ],
        scratch_types=[
            bufset(), bufset(),
            pltpu.SemaphoreType.DMA, pltpu.SemaphoreType.DMA,
            pltpu.SemaphoreType.DMA, pltpu.SemaphoreType.DMA,
        ],
    )
    return f(A, B, edge_src, edge_tgt)


_GCH = 2000
_NGCH = _EPW // _GCH

def _geo_body(psx_h, psy_h, psz_h, ptx_h, pty_h, ptz_h, es_hbm, et_hbm,
              sq_out, ptabs, idx_s, idx_t, sqb):
    wid = lax.axis_index("s") * 2 + lax.axis_index("c")
    tbase = wid * _EPW

    psx_t, psy_t, psz_t, ptx_t, pty_t, ptz_t = ptabs
    pltpu.sync_copy(psx_h, psx_t)
    pltpu.sync_copy(psy_h, psy_t)
    pltpu.sync_copy(psz_h, psz_t)
    pltpu.sync_copy(ptx_h, ptx_t)
    pltpu.sync_copy(pty_h, pty_t)
    pltpu.sync_copy(ptz_h, ptz_t)

    def body(i, carry):
        base = tbase + i * _GCH
        pltpu.sync_copy(es_hbm.at[pl.ds(base, _GCH)], idx_s)
        pltpu.sync_copy(et_hbm.at[pl.ds(base, _GCH)], idx_t)

        def grp(g, carry2):
            sl = pl.ds(g * 16, 16)
            i_s = idx_s[sl]
            i_t = idx_t[sl]
            dx = plsc.load_gather(ptx_t, [i_t]) - plsc.load_gather(psx_t, [i_s])
            dy = plsc.load_gather(pty_t, [i_t]) - plsc.load_gather(psy_t, [i_s])
            dz = plsc.load_gather(ptz_t, [i_t]) - plsc.load_gather(psz_t, [i_s])
            sqb[sl] = dx * dx + dy * dy + dz * dz
            return carry2

        lax.fori_loop(0, _GCH // 16, grp, 0)
        pltpu.sync_copy(sqb, sq_out.at[pl.ds(base, _GCH)])
        return carry

    lax.fori_loop(0, _NGCH, body, 0)


def _sc_geo(psx, psy, psz, ptx, pty, ptz, edge_src, edge_tgt):
    f = pl.kernel(
        _geo_body, mesh=_sc_mesh(),
        compiler_params=pltpu.CompilerParams(needs_layout_passes=False),
        out_type=[jax.ShapeDtypeStruct((E_TOT,), jnp.float32)],
        scratch_types=[
            [pltpu.VMEM((N_NODE,), jnp.float32) for _ in range(6)],
            pltpu.VMEM((_GCH,), jnp.int32),
            pltpu.VMEM((_GCH,), jnp.int32),
            pltpu.VMEM((_GCH,), jnp.float32),
        ],
    )
    return f(psx, psy, psz, ptx, pty, ptz, edge_src, edge_tgt)[0]


def _silu(x):
    return x * jax.nn.sigmoid(x)


def _scatter_body(msg_hbm, et_hbm, zn_hbm, agg_out,
                  bufs0, bufs1, agg_acc, lsem0, lsem1):
    c = lax.axis_index("c")
    s = lax.axis_index("s")
    wid = s * 2 + c
    tbase = wid * _EPW

    @pl.when(s == 0)
    def _init():
        pltpu.sync_copy(zn_hbm, agg_acc)

    def fire_l(base, bufs, sem):
        idx_t, mb = bufs
        pltpu.sync_copy(et_hbm.at[pl.ds(base, _CH)], idx_t)
        pltpu.async_copy(msg_hbm.at[pl.ds(base, _CH)], mb, sem)

    def wait_l(bufs, sem):
        pltpu.make_async_copy(msg_hbm.at[pl.ds(0, _CH)], bufs[1], sem).wait()

    def scat(bufs):
        idx_t, mb = bufs
        pltpu.sync_copy(mb, agg_acc.at[idx_t], add=True)

    plsc.subcore_barrier()
    fire_l(tbase, bufs0, lsem0)

    def body(j, carry):
        base1 = tbase + (2 * j + 1) * _CH
        base2 = tbase + (2 * j + 2) * _CH

        @pl.when(2 * j + 1 < _NCH)
        def _():
            fire_l(base1, bufs1, lsem1)

        wait_l(bufs0, lsem0)
        scat(bufs0)

        @pl.when(2 * j + 2 < _NCH)
        def _():
            fire_l(base2, bufs0, lsem0)

        @pl.when(2 * j + 1 < _NCH)
        def _():
            wait_l(bufs1, lsem1)
            scat(bufs1)

        return carry

    lax.fori_loop(0, (_NCH + 1) // 2, body, 0)
    plsc.subcore_barrier()

    rows = N_NODE // 16
    rbase = s * rows
    pltpu.sync_copy(agg_acc.at[pl.ds(rbase, rows)],
                    agg_out.at[c].at[pl.ds(rbase, rows)])


def _sc_scatter(msg, edge_tgt, zn):
    f = pl.kernel(
        _scatter_body, mesh=_sc_mesh(),
        compiler_params=pltpu.CompilerParams(use_tc_tiling_on_sc=False),
        out_type=[
            jax.ShapeDtypeStruct((2, N_NODE, C), jnp.float32),
        ],
        scratch_types=[
            [
                pltpu.VMEM((_CH,), jnp.int32),
                pltpu.VMEM((_CH, C), jnp.float32),
            ],
            [
                pltpu.VMEM((_CH,), jnp.int32),
                pltpu.VMEM((_CH, C), jnp.float32),
            ],
            pltpu.VMEM_SHARED((N_NODE, C), jnp.float32),
            pltpu.SemaphoreType.DMA, pltpu.SemaphoreType.DMA,
        ],
    )
    return f(msg, edge_tgt, zn)[0]


def _vel_body(w_hbm, psx_h, psy_h, psz_h, ptx_h, pty_h, ptz_h,
              es_hbm, et_hbm, zv_hbm,
              vx_out, vy_out, vz_out,
              bufs0, bufs1, ptabs, accs, lsem0, lsem1):
    c = lax.axis_index("c")
    s = lax.axis_index("s")
    wid = s * 2 + c
    tbase = wid * _EPW
    vx_acc, vy_acc, vz_acc = accs

    psx_t, psy_t, psz_t, ptx_t, pty_t, ptz_t = ptabs
    pltpu.sync_copy(psx_h, psx_t)
    pltpu.sync_copy(psy_h, psy_t)
    pltpu.sync_copy(psz_h, psz_t)
    pltpu.sync_copy(ptx_h, ptx_t)
    pltpu.sync_copy(pty_h, pty_t)
    pltpu.sync_copy(ptz_h, ptz_t)

    @pl.when(s == 0)
    def _init():
        pltpu.sync_copy(zv_hbm, vx_acc)
        pltpu.sync_copy(zv_hbm, vy_acc)
        pltpu.sync_copy(zv_hbm, vz_acc)

    def fire_l(base, bufs, sem):
        idx_s, idx_t, wb, xb, yb, zb = bufs
        pltpu.sync_copy(es_hbm.at[pl.ds(base, _CH)], idx_s)
        pltpu.sync_copy(et_hbm.at[pl.ds(base, _CH)], idx_t)
        pltpu.async_copy(w_hbm.at[pl.ds(base, _CH)], wb, sem)

    def wait_l(bufs, sem):
        pltpu.make_async_copy(w_hbm.at[pl.ds(0, _CH)], bufs[2], sem).wait()

    def scat(bufs):
        idx_s, idx_t, wb, xb, yb, zb = bufs
        for g in range(_CH // 16):
            sl = pl.ds(g * 16, 16)
            i_s = idx_s[sl]
            i_t = idx_t[sl]
            wv = wb[sl]
            xb[sl] = wv * (plsc.load_gather(ptx_t, [i_t])
                           - plsc.load_gather(psx_t, [i_s]))
            yb[sl] = wv * (plsc.load_gather(pty_t, [i_t])
                           - plsc.load_gather(psy_t, [i_s]))
            zb[sl] = wv * (plsc.load_gather(ptz_t, [i_t])
                           - plsc.load_gather(psz_t, [i_s]))
        pltpu.sync_copy(xb, vx_acc.at[idx_t], add=True)
        pltpu.sync_copy(yb, vy_acc.at[idx_t], add=True)
        pltpu.sync_copy(zb, vz_acc.at[idx_t], add=True)

    plsc.subcore_barrier()
    fire_l(tbase, bufs0, lsem0)

    def body(j, carry):
        base1 = tbase + (2 * j + 1) * _CH
        base2 = tbase + (2 * j + 2) * _CH

        @pl.when(2 * j + 1 < _NCH)
        def _():
            fire_l(base1, bufs1, lsem1)

        wait_l(bufs0, lsem0)
        scat(bufs0)

        @pl.when(2 * j + 2 < _NCH)
        def _():
            fire_l(base2, bufs0, lsem0)

        @pl.when(2 * j + 1 < _NCH)
        def _():
            wait_l(bufs1, lsem1)
            scat(bufs1)

        return carry

    lax.fori_loop(0, (_NCH + 1) // 2, body, 0)
    plsc.subcore_barrier()

    rows = N_NODE // 10
    rbase = s * rows

    @pl.when(s < 10)
    def _dump():
        pltpu.sync_copy(vx_acc.at[pl.ds(rbase, rows)],
                        vx_out.at[c].at[pl.ds(rbase, rows)])
        pltpu.sync_copy(vy_acc.at[pl.ds(rbase, rows)],
                        vy_out.at[c].at[pl.ds(rbase, rows)])
        pltpu.sync_copy(vz_acc.at[pl.ds(rbase, rows)],
                        vz_out.at[c].at[pl.ds(rbase, rows)])


def _sc_vel_scatter(w, psx, psy, psz, ptx, pty, ptz, edge_src, edge_tgt, zv):
    bufset = lambda: [
        pltpu.VMEM((_CH,), jnp.int32),
        pltpu.VMEM((_CH,), jnp.int32),
        pltpu.VMEM((_CH,), jnp.float32),
        pltpu.VMEM((_CH,), jnp.float32),
        pltpu.VMEM((_CH,), jnp.float32),
        pltpu.VMEM((_CH,), jnp.float32),
    ]
    f = pl.kernel(
        _vel_body, mesh=_sc_mesh(),
        compiler_params=pltpu.CompilerParams(
            use_tc_tiling_on_sc=False, needs_layout_passes=False),
        out_type=[
            jax.ShapeDtypeStruct((2, N_NODE), jnp.float32),
            jax.ShapeDtypeStruct((2, N_NODE), jnp.float32),
            jax.ShapeDtypeStruct((2, N_NODE), jnp.float32),
        ],
        scratch_types=[
            bufset(), bufset(),
            [pltpu.VMEM((N_NODE,), jnp.float32) for _ in range(6)],
            [pltpu.VMEM_SHARED((N_NODE,), jnp.float32) for _ in range(3)],
            pltpu.SemaphoreType.DMA, pltpu.SemaphoreType.DMA,
        ],
    )
    return f(w, psx, psy, psz, ptx, pty, ptz, edge_src, edge_tgt, zv)


def _pre_body(h_src_ref, h_tgt_ref, t_emb_ref, w1s_ref, w1t_ref, w1e_ref,
              b1_ref, a_ref, b_ref):
    a_ref[...] = jnp.dot(h_src_ref[...], w1s_ref[...],
                         preferred_element_type=jnp.float32)
    b_ref[...] = (jnp.dot(h_tgt_ref[...], w1t_ref[...],
                          preferred_element_type=jnp.float32)
                  + jnp.dot(t_emb_ref[...], w1e_ref[...],
                            preferred_element_type=jnp.float32)
                  + b1_ref[...])


def _node_pre(h_src, h_tgt, t_emb, w1s, w1t, w1e, b1):
    grid = N_NODE // _NB
    return pl.pallas_call(
        _pre_body,
        grid=(grid,),
        in_specs=[
            pl.BlockSpec((_NB, C), lambda i: (i, 0)),
            pl.BlockSpec((_NB, C), lambda i: (i, 0)),
            pl.BlockSpec((_NB, 32), lambda i: (i, 0)),
            pl.BlockSpec((C, 2 * C), lambda i: (0, 0)),
            pl.BlockSpec((C, 2 * C), lambda i: (0, 0)),
            pl.BlockSpec((32, 2 * C), lambda i: (0, 0)),
            pl.BlockSpec((1, 2 * C), lambda i: (0, 0)),
        ],
        out_specs=[
            pl.BlockSpec((_NB, 2 * C), lambda i: (i, 0)),
            pl.BlockSpec((_NB, 2 * C), lambda i: (i, 0)),
        ],
        out_shape=[
            jax.ShapeDtypeStruct((N_NODE, 2 * C), jnp.float32),
            jax.ShapeDtypeStruct((N_NODE, 2 * C), jnp.float32),
        ],
    )(h_src, h_tgt, t_emb, w1s, w1t, w1e, b1)


def _edge_body(a_ref, b_ref, sq_ref,
               w1d_ref, w2_ref, b2_ref, wc1_ref, bc1_ref, wc2_ref, bc2_ref,
               msg_ref, w_ref):
    sq2 = sq_ref[...].reshape(1, _EB)
    sq_term = lax.dot_general(sq2, w1d_ref[...], (((0,), (0,)), ((), ())),
                              preferred_element_type=jnp.float32)
    pre = a_ref[...] + b_ref[...] + sq_term
    h1 = _silu(pre)
    msg = _silu(jnp.dot(h1, w2_ref[...], preferred_element_type=jnp.float32)
                + b2_ref[...])
    t1 = _silu(jnp.dot(msg, wc1_ref[...], preferred_element_type=jnp.float32)
               + bc1_ref[...])
    w = jnp.tanh(lax.dot_general(wc2_ref[...], t1, (((0,), (1,)), ((), ())),
                 preferred_element_type=jnp.float32) + bc2_ref[...])
    msg_ref[...] = msg
    w_ref[...] = w.reshape(_EB)


def _edge_mlp(a_rows, b_rows, sq, w1d, W2, b2, Wc1, bc1, Wc2, bc2):
    grid = E_TOT // _EB
    vec = lambda: pl.BlockSpec((_EB,), lambda i: (i,))
    return pl.pallas_call(
        _edge_body,
        grid=(grid,),
        in_specs=[
            pl.BlockSpec((_EB, 2 * C), lambda i: (i, 0)),
            pl.BlockSpec((_EB, 2 * C), lambda i: (i, 0)),
            vec(),
            pl.BlockSpec((1, 2 * C), lambda i: (0, 0)),
            pl.BlockSpec((2 * C, C), lambda i: (0, 0)),
            pl.BlockSpec((1, C), lambda i: (0, 0)),
            pl.BlockSpec((C, C // 2), lambda i: (0, 0)),
            pl.BlockSpec((1, C // 2), lambda i: (0, 0)),
            pl.BlockSpec((C // 2, 1), lambda i: (0, 0)),
            pl.BlockSpec((1, 1), lambda i: (0, 0)),
        ],
        out_specs=[
            pl.BlockSpec((_EB, C), lambda i: (i, 0)),
            vec(),
        ],
        out_shape=[
            jax.ShapeDtypeStruct((E_TOT, C), jnp.float32),
            jax.ShapeDtypeStruct((E_TOT,), jnp.float32),
        ],
    )(a_rows, b_rows, sq, w1d, W2, b2, Wc1, bc1, Wc2, bc2)


def _upd_body(h_ref, agg0_ref, agg1_ref, vx0_ref, vx1_ref, vy0_ref, vy1_ref,
              vz0_ref, vz1_ref, wu1a_ref, wu1b_ref, bu1_ref, wu2_ref,
              bu2_ref, g_ref, bt_ref,
              h_out_ref, vx_ref, vy_ref, vz_ref):
    h = h_ref[...]
    agg = agg0_ref[...] + agg1_ref[...]
    u1 = _silu(jnp.dot(h, wu1a_ref[...], preferred_element_type=jnp.float32)
               + jnp.dot(agg, wu1b_ref[...], preferred_element_type=jnp.float32)
               + bu1_ref[...])
    upd = jnp.dot(u1, wu2_ref[...], preferred_element_type=jnp.float32) + bu2_ref[...]
    x = h + upd
    mu = jnp.mean(x, axis=1, keepdims=True)
    xc = x - mu
    var = jnp.mean(xc * xc, axis=1, keepdims=True)
    h_out_ref[...] = xc * lax.rsqrt(var + 1e-5) * g_ref[...] + bt_ref[...]
    vx_ref[...] = vx0_ref[...] + vx1_ref[...]
    vy_ref[...] = vy0_ref[...] + vy1_ref[...]
    vz_ref[...] = vz0_ref[...] + vz1_ref[...]


def _node_update(h_tgt, agg0, agg1, vx0, vx1, vy0, vy1, vz0, vz1,
                 Wu1a, Wu1b, bu1, Wu2, bu2, gamma, beta):
    grid = N_NODE // _NB
    vec = lambda: pl.BlockSpec((N_NODE,), lambda i: (0,))
    return pl.pallas_call(
        _upd_body,
        grid=(grid,),
        in_specs=[
            pl.BlockSpec((_NB, C), lambda i: (i, 0)),
            pl.BlockSpec((_NB, C), lambda i: (i, 0)),
            pl.BlockSpec((_NB, C), lambda i: (i, 0)),
            vec(), vec(), vec(), vec(), vec(), vec(),
            pl.BlockSpec((C, C), lambda i: (0, 0)),
            pl.BlockSpec((C, C), lambda i: (0, 0)),
            pl.BlockSpec((1, C), lambda i: (0, 0)),
            pl.BlockSpec((C, C), lambda i: (0, 0)),
            pl.BlockSpec((1, C), lambda i: (0, 0)),
            pl.BlockSpec((1, C), lambda i: (0, 0)),
            pl.BlockSpec((1, C), lambda i: (0, 0)),
        ],
        out_specs=[
            pl.BlockSpec((_NB, C), lambda i: (i, 0)),
            vec(), vec(), vec(),
        ],
        out_shape=[
            jax.ShapeDtypeStruct((N_NODE, C), jnp.float32),
            jax.ShapeDtypeStruct((N_NODE,), jnp.float32),
            jax.ShapeDtypeStruct((N_NODE,), jnp.float32),
            jax.ShapeDtypeStruct((N_NODE,), jnp.float32),
        ],
    )(h_tgt, agg0, agg1, vx0, vx1, vy0, vy1, vz0, vz1,
      Wu1a, Wu1b, bu1, Wu2, bu2, gamma, beta)


def kernel(h_src, h_tgt, pos_src, pos_tgt, t_emb_tgt, edge_src, edge_tgt,
           W1, b1, W2, b2, Wc1, bc1, Wc2, bc2, Wu1, bu1, Wu2, bu2,
           gamma, beta):
    w1s = W1[0:C]
    w1t = W1[C:2 * C]
    w1d = W1[2 * C:2 * C + 1]
    w1e = W1[2 * C + 1:]
    A, B = _node_pre(h_src, h_tgt, t_emb_tgt, w1s, w1t, w1e,
                     b1.reshape(1, -1))

    psx, psy, psz = pos_src[:, 0], pos_src[:, 1], pos_src[:, 2]
    ptx, pty, ptz = pos_tgt[:, 0], pos_tgt[:, 1], pos_tgt[:, 2]

    a_rows, b_rows = _sc_gather(A, B, edge_src, edge_tgt)
    sq = _sc_geo(psx, psy, psz, ptx, pty, ptz, edge_src, edge_tgt)

    msg, w = _edge_mlp(a_rows, b_rows, sq, w1d, W2,
                       b2.reshape(1, -1), Wc1, bc1.reshape(1, -1),
                       Wc2, bc2.reshape(1, 1))

    agg2 = _sc_scatter(msg, edge_tgt, jnp.zeros((N_NODE, C), jnp.float32))
    vx2, vy2, vz2 = _sc_vel_scatter(w, psx, psy, psz, ptx, pty, ptz,
                                    edge_src, edge_tgt,
                                    jnp.zeros((N_NODE,), jnp.float32))

    h_new, vx, vy, vz = _node_update(
        h_tgt, agg2[0], agg2[1], vx2[0], vx2[1], vy2[0], vy2[1],
        vz2[0], vz2[1], Wu1[:C], Wu1[C:], bu1.reshape(1, -1), Wu2,
        bu2.reshape(1, -1), gamma.reshape(1, -1), beta.reshape(1, -1))
    return h_new, jnp.stack([vx, vy, vz], axis=1)
